# Initial kernel scaffold; baseline (speedup 1.0000x reference)
import jax
import jax.numpy as jnp
from jax.experimental import pallas as pl


def kernel(inputs, hidden_state, edge_index, W1, b1, W_ih, W_hh, b_ih, b_hh, W2, b2, Wg1, bg1, Wg2, bg2):
    raise NotImplementedError("write your pallas kernel here")


# speedup vs baseline: 23.1819x; 23.1819x over previous
import functools

import jax
import jax.numpy as jnp
from jax import lax
from jax.experimental import pallas as pl
from jax.experimental.pallas import tpu as pltpu
from jax.experimental.pallas import tpu_sc as plsc

N = 50000
NE = 800000
HF = 64
PW = 32
NC, NSUB = 2, 16
EROWS = NE // 128

_MESH = plsc.VectorSubcoreMesh(
    core_axis_name="c", subcore_axis_name="s", num_cores=NC, num_subcores=NSUB
)

_Z16 = lambda: jnp.zeros((16,), jnp.float32)


def _fill_zeros_2d(ref, nrows):
    def body(k, _):
        ref[k >> 1, pl.ds((k & 1) * 16, 16)] = _Z16()
        return 0
    lax.fori_loop(0, nrows * 2, body, 0)


def _deg_body(dst_hbm, out_hbm, stg, ones_v, idx_s, acc):
    c = lax.axis_index("c")
    s = lax.axis_index("s")

    def zfill(i, _):
        stg[pl.ds(jnp.minimum(i * 16, 5000 - 16), 16)] = _Z16()
        return 0
    lax.fori_loop(0, 313, zfill, 0)

    def ofill(i, _):
        ones_v[pl.ds(i * 16, 16)] = jnp.ones((16,), jnp.float32)
        return 0
    lax.fori_loop(0, 8, ofill, 0)

    @pl.when(s < 10)
    def _():
        pltpu.sync_copy(stg, acc.at[pl.ds(s * 5000, 5000)])
    plsc.subcore_barrier()

    start = c * 3125 + s * 195

    def stage(ko, _):
        pltpu.sync_copy(dst_hbm.at[pl.ds(start + ko * 15, 15)], idx_s)
        def row(jj, _):
            pltpu.sync_copy(ones_v, acc.at[idx_s.at[jj]], add=True)
            return 0
        lax.fori_loop(0, 15, row, 0)
        return 0
    lax.fori_loop(0, 13, stage, 0)

    @pl.when(s < 5)
    def _():
        pltpu.sync_copy(dst_hbm.at[c * 3125 + 3120 + s], idx_s.at[0])
        pltpu.sync_copy(ones_v, acc.at[idx_s.at[0]], add=True)

    plsc.subcore_barrier()

    @pl.when(s < 10)
    def _():
        pltpu.sync_copy(acc.at[pl.ds(s * 5000, 5000)], stg)
        pltpu.sync_copy(stg, out_hbm.at[c, pl.ds(s * 5000, 5000)])


_SC_PARAMS = pltpu.CompilerParams(use_tc_tiling_on_sc=False)

_deg_kernel = functools.partial(
    pl.kernel,
    out_type=jax.ShapeDtypeStruct((NC, N), jnp.float32),
    mesh=_MESH,
    compiler_params=_SC_PARAMS,
    scratch_types=[
        pltpu.VMEM((5000,), jnp.float32),
        pltpu.VMEM((128,), jnp.float32),
        pltpu.VMEM((15, 128), jnp.int32),
        pltpu.VMEM_SHARED((N,), jnp.float32),
    ],
)(_deg_body)


def _zero_acc_and_barrier(stg, acc, s):
    _fill_zeros_2d(stg, 625)

    @pl.when(s < 10)
    def _():
        def zc(t, _):
            pltpu.sync_copy(stg, acc.at[pl.ds(s * 5000 + t * 625, 625)])
            return 0
        lax.fori_loop(0, 8, zc, 0)
    plsc.subcore_barrier()


def _writeback(stg, acc, out_row, s):
    plsc.subcore_barrier()

    @pl.when(s < 10)
    def _():
        def wb(t, _):
            sl = pl.ds(s * 5000 + t * 625, 625)
            pltpu.sync_copy(acc.at[sl], stg)
            pltpu.sync_copy(stg, out_row.at[sl])
            return 0
        lax.fori_loop(0, 8, wb, 0)


def _edge_block(tbl_row, acc, rows, sem, idx_s, idx_d):
    def row(jj, _):
        pltpu.async_copy(tbl_row.at[idx_s.at[jj]], rows, sem).wait()
        pltpu.sync_copy(rows, acc.at[idx_d.at[jj]], add=True)
        return 0
    lax.fori_loop(0, 15, row, 0)


def _gcn1_body(src_hbm, dst_hbm, tbl_hbm, out_hbm, stg, rows, idx_s, idx_d, acc, sem):
    c = lax.axis_index("c")
    s = lax.axis_index("s")
    _zero_acc_and_barrier(stg, acc, s)

    tbl_row = tbl_hbm.at[c]
    start = s * 390

    def stage(ko, _):
        sl = pl.ds(start + ko * 15, 15)
        pltpu.sync_copy(src_hbm.at[sl], idx_s)
        pltpu.sync_copy(dst_hbm.at[sl], idx_d)
        _edge_block(tbl_row, acc, rows, sem, idx_s, idx_d)
        return 0
    lax.fori_loop(0, 26, stage, 0)

    @pl.when(s < 10)
    def _():
        row = 6240 + s
        pltpu.sync_copy(src_hbm.at[row], idx_s.at[0])
        pltpu.sync_copy(dst_hbm.at[row], idx_d.at[0])
        pltpu.async_copy(tbl_row.at[idx_s.at[0]], rows, sem).wait()
        pltpu.sync_copy(rows, acc.at[idx_d.at[0]], add=True)

    _writeback(stg, acc, out_hbm.at[c], s)


def _gcn2_body(src_hbm, dst_hbm, tbl_hbm, out_hbm, stg, rows, idx_s, idx_d, acc, sem):
    c = lax.axis_index("c")
    s = lax.axis_index("s")
    _zero_acc_and_barrier(stg, acc, s)

    start = c * 3125 + s * 195

    def stage(ko, _):
        sl = pl.ds(start + ko * 15, 15)
        pltpu.sync_copy(src_hbm.at[sl], idx_s)
        pltpu.sync_copy(dst_hbm.at[sl], idx_d)
        _edge_block(tbl_hbm, acc, rows, sem, idx_s, idx_d)
        return 0
    lax.fori_loop(0, 13, stage, 0)

    @pl.when(s < 5)
    def _():
        row = c * 3125 + 3120 + s
        pltpu.sync_copy(src_hbm.at[row], idx_s.at[0])
        pltpu.sync_copy(dst_hbm.at[row], idx_d.at[0])
        pltpu.async_copy(tbl_hbm.at[idx_s.at[0]], rows, sem).wait()
        pltpu.sync_copy(rows, acc.at[idx_d.at[0]], add=True)

    _writeback(stg, acc, out_hbm.at[c], s)


_SC_SCRATCH = [
    pltpu.VMEM((625, PW), jnp.float32),
    pltpu.VMEM((128, PW), jnp.float32),
    pltpu.VMEM((15, 128), jnp.int32),
    pltpu.VMEM((15, 128), jnp.int32),
    pltpu.VMEM_SHARED((N, PW), jnp.float32),
    pltpu.SemaphoreType.DMA,
]

_gcn1_kernel = functools.partial(
    pl.kernel,
    out_type=jax.ShapeDtypeStruct((NC, N, PW), jnp.float32),
    mesh=_MESH,
    compiler_params=_SC_PARAMS,
    scratch_types=_SC_SCRATCH,
)(_gcn1_body)

_gcn2_kernel = functools.partial(
    pl.kernel,
    out_type=jax.ShapeDtypeStruct((NC, N, PW), jnp.float32),
    mesh=_MESH,
    compiler_params=_SC_PARAMS,
    scratch_types=_SC_SCRATCH,
)(_gcn2_body)


RB = 2000
GRID = N // RB


def _dinv_from(degr):
    deg = degr[:, 0] + degr[:, 1] + 1.0
    return lax.rsqrt(deg)[:, None]


def _dense_body(x_r, h0_r, w1_r, b1_r, wih_r, bih_r, whh_r, bhh_r, w2_r, b2_r,
                deg_r, h_o, m_o):
    x = jnp.maximum(jnp.dot(x_r[...], w1_r[...],
                            preferred_element_type=jnp.float32) + b1_r[...], 0.0)
    gi = jnp.dot(x, wih_r[...], preferred_element_type=jnp.float32) + bih_r[...]
    gh = jnp.dot(h0_r[...], whh_r[...], preferred_element_type=jnp.float32) + bhh_r[...]
    r = jax.nn.sigmoid(gi[:, :HF] + gh[:, :HF])
    z = jax.nn.sigmoid(gi[:, HF:2 * HF] + gh[:, HF:2 * HF])
    n = jnp.tanh(gi[:, 2 * HF:] + r * gh[:, 2 * HF:])
    h = (1.0 - z) * n + z * h0_r[...]
    o = jnp.dot(h, w2_r[...], preferred_element_type=jnp.float32) + b2_r[...]
    m = o * _dinv_from(deg_r[...])
    h_o[...] = h
    m_o[...] = jnp.stack([m[:, :PW], m[:, PW:]], axis=0)


def _tc_dense(x2d, h02d, w1, b1, wihT, bih, whhT, bhh, w2, b2, degp):
    full = lambda shape: pl.BlockSpec(shape, lambda i: (0,) * len(shape))
    return pl.pallas_call(
        _dense_body,
        grid=(GRID,),
        in_specs=[
            pl.BlockSpec((RB, 128), lambda i: (i, 0)),
            pl.BlockSpec((RB, HF), lambda i: (i, 0)),
            full((128, HF)), full((1, HF)),
            full((HF, 3 * HF)), full((1, 3 * HF)),
            full((HF, 3 * HF)), full((1, 3 * HF)),
            full((HF, HF)), full((1, HF)),
            pl.BlockSpec((RB, NC), lambda i: (i, 0)),
        ],
        out_specs=[
            pl.BlockSpec((RB, HF), lambda i: (i, 0)),
            pl.BlockSpec((NC, RB, PW), lambda i: (0, i, 0)),
        ],
        out_shape=[
            jax.ShapeDtypeStruct((N, HF), jnp.float32),
            jax.ShapeDtypeStruct((NC, N, PW), jnp.float32),
        ],
    )(x2d, h02d, w1, b1, wihT, bih, whhT, bhh, w2, b2, degp)


def _mid_body(acc_r, m_r, deg_r, wg1_r, bg1_r, wg2_r, out_r):
    dinv = _dinv_from(deg_r[...])
    s = jnp.concatenate([acc_r[0] + m_r[0], acc_r[1] + m_r[1]], axis=1) * dinv
    q1 = jnp.maximum(jnp.dot(s, wg1_r[...],
                             preferred_element_type=jnp.float32) + bg1_r[...], 0.0)
    out_r[...] = jnp.dot(q1, wg2_r[...], preferred_element_type=jnp.float32) * dinv


def _tc_mid(acc1, mboth, degp, wg1, bg1, wg2p):
    full = lambda shape: pl.BlockSpec(shape, lambda i: (0,) * len(shape))
    return pl.pallas_call(
        _mid_body,
        grid=(GRID,),
        in_specs=[
            pl.BlockSpec((NC, RB, PW), lambda i: (0, i, 0)),
            pl.BlockSpec((NC, RB, PW), lambda i: (0, i, 0)),
            pl.BlockSpec((RB, NC), lambda i: (i, 0)),
            full((HF, HF)), full((1, HF)), full((HF, PW)),
        ],
        out_specs=pl.BlockSpec((RB, PW), lambda i: (i, 0)),
        out_shape=jax.ShapeDtypeStruct((N, PW), jnp.float32),
    )(acc1, mboth, degp, wg1, bg1, wg2p)


def _final_body(acc_r, m_r, deg_r, bg2_r, out_r):
    dinv = _dinv_from(deg_r[...])
    out_r[...] = (acc_r[0] + acc_r[1] + m_r[...]) * dinv + bg2_r[...]


def _tc_final(acc2, m2p, degp, bg2p):
    full = lambda shape: pl.BlockSpec(shape, lambda i: (0,) * len(shape))
    return pl.pallas_call(
        _final_body,
        grid=(GRID,),
        in_specs=[
            pl.BlockSpec((NC, RB, PW), lambda i: (0, i, 0)),
            pl.BlockSpec((RB, PW), lambda i: (i, 0)),
            pl.BlockSpec((RB, NC), lambda i: (i, 0)),
            full((1, PW)),
        ],
        out_specs=pl.BlockSpec((RB, PW), lambda i: (i, 0)),
        out_shape=jax.ShapeDtypeStruct((N, PW), jnp.float32),
    )(acc2, m2p, degp, bg2p)


def kernel(inputs, hidden_state, edge_index, W1, b1, W_ih, W_hh, b_ih, b_hh,
           W2, b2, Wg1, bg1, Wg2, bg2):
    b, a, e = inputs.shape
    x2d = inputs.reshape(N, e)
    h02d = hidden_state.reshape(N, HF)
    src2d = edge_index[0].reshape(EROWS, 128)
    dst2d = edge_index[1].reshape(EROWS, 128)

    wihT = W_ih.T
    whhT = W_hh.T
    wg2p = jnp.pad(Wg2, ((0, 0), (0, PW - Wg2.shape[1])))
    bg2p = jnp.pad(bg2, (0, PW - bg2.shape[0])).reshape(1, PW)
    row = lambda v: v.reshape(1, -1)

    degp = _deg_kernel(dst2d).T
    h, mboth = _tc_dense(x2d, h02d, W1, row(b1), wihT, row(b_ih), whhT,
                         row(b_hh), W2, row(b2), degp)
    acc1 = _gcn1_kernel(src2d, dst2d, mboth)
    m2p = _tc_mid(acc1, mboth, degp, Wg1, row(bg1), wg2p)
    acc2 = _gcn2_kernel(src2d, dst2d, m2p)
    q32 = _tc_final(acc2, m2p, degp, bg2p)
    return q32[:, :20].reshape(b, a, 20), h.reshape(b, a, HF)


# speedup vs baseline: 27.4899x; 1.1858x over previous
import functools

import jax
import jax.numpy as jnp
from jax import lax
from jax.experimental import pallas as pl
from jax.experimental.pallas import tpu as pltpu
from jax.experimental.pallas import tpu_sc as plsc

N = 50000
NE = 800000
HF = 64
PW = 16
NC, NSUB = 2, 16
EW = 1000

_MESH = plsc.VectorSubcoreMesh(
    core_axis_name="c", subcore_axis_name="s", num_cores=NC, num_subcores=NSUB
)
_SC_PARAMS = pltpu.CompilerParams(use_tc_tiling_on_sc=False)

_Z16 = lambda: jnp.zeros((16,), jnp.float32)


def _deg_body(dst_hbm, out_hbm, stg, ones_v, idx_d, acc):
    c = lax.axis_index("c")
    s = lax.axis_index("s")

    def zfill(i, _):
        stg[pl.ds(jnp.minimum(i * 16, 5000 - 16), 16)] = _Z16()
        return 0
    lax.fori_loop(0, 313, zfill, 0)

    def ofill(i, _):
        ones_v[pl.ds(i * 16, 16)] = jnp.ones((16,), jnp.float32)
        return 0
    lax.fori_loop(0, EW // 16, ofill, 0)

    @pl.when(s < 10)
    def _():
        pltpu.sync_copy(stg, acc.at[pl.ds(s * 5000, 5000)])
    plsc.subcore_barrier()

    base = (c * NSUB + s) * (NE // (NC * NSUB))

    def win(w, _):
        pltpu.sync_copy(dst_hbm.at[pl.ds(base + w * EW, EW)], idx_d)
        pltpu.sync_copy(ones_v, acc.at[idx_d], add=True)
        return 0
    lax.fori_loop(0, 25000 // EW, win, 0)

    plsc.subcore_barrier()

    @pl.when(s < 10)
    def _():
        pltpu.sync_copy(acc.at[pl.ds(s * 5000, 5000)], stg)
        pltpu.sync_copy(stg, out_hbm.at[c, pl.ds(s * 5000, 5000)])


_deg_kernel = functools.partial(
    pl.kernel,
    out_type=jax.ShapeDtypeStruct((NC, N), jnp.float32),
    mesh=_MESH,
    compiler_params=_SC_PARAMS,
    scratch_types=[
        pltpu.VMEM((5000,), jnp.float32),
        pltpu.VMEM((EW,), jnp.float32),
        pltpu.VMEM((EW,), jnp.int32),
        pltpu.VMEM_SHARED((N,), jnp.float32),
    ],
)(_deg_body)


def _zero_acc_and_barrier(stg, acc, s):
    def body(k, _):
        stg[k, pl.ds(0, 16)] = _Z16()
        return 0
    lax.fori_loop(0, 625, body, 0)

    @pl.when(s < 10)
    def _():
        def zc(t, _):
            pltpu.sync_copy(stg, acc.at[pl.ds(s * 5000 + t * 625, 625)])
            return 0
        lax.fori_loop(0, 8, zc, 0)
    plsc.subcore_barrier()


def _writeback(stg, acc, out_row, s):
    plsc.subcore_barrier()

    @pl.when(s < 10)
    def _():
        def wb(t, _):
            sl = pl.ds(s * 5000 + t * 625, 625)
            pltpu.sync_copy(acc.at[sl], stg)
            pltpu.sync_copy(stg, out_row.at[sl])
            return 0
        lax.fori_loop(0, 8, wb, 0)


def _edge_pipeline(src_hbm, dst_hbm, tbl_row, acc, base, nwin,
                   rows_a, rows_b, is_a, id_a, is_b, id_b, sem_a, sem_b):
    def stage(w, idx_s, idx_d):
        sl = pl.ds(base + w * EW, EW)
        pltpu.sync_copy(src_hbm.at[sl], idx_s)
        pltpu.sync_copy(dst_hbm.at[sl], idx_d)

    def gather(idx_s, rows, sem):
        return pltpu.async_copy(tbl_row.at[idx_s], rows, sem)

    def wait(idx_s, rows, sem):
        pltpu.make_async_copy(tbl_row.at[idx_s], rows, sem).wait()

    def scatter(rows, idx_d):
        pltpu.sync_copy(rows, acc.at[idx_d], add=True)

    stage(0, is_a, id_a)
    gather(is_a, rows_a, sem_a)

    def pair(k, _):
        w = 1 + 2 * k
        stage(w, is_b, id_b)
        gather(is_b, rows_b, sem_b)
        wait(is_a, rows_a, sem_a)
        scatter(rows_a, id_a)
        stage(w + 1, is_a, id_a)
        gather(is_a, rows_a, sem_a)
        wait(is_b, rows_b, sem_b)
        scatter(rows_b, id_b)
        return 0
    lax.fori_loop(0, (nwin - 1) // 2, pair, 0)

    if nwin % 2 == 0:
        stage(nwin - 1, is_b, id_b)
        gather(is_b, rows_b, sem_b)
        wait(is_a, rows_a, sem_a)
        scatter(rows_a, id_a)
        wait(is_b, rows_b, sem_b)
        scatter(rows_b, id_b)
    else:
        wait(is_a, rows_a, sem_a)
        scatter(rows_a, id_a)


def _make_gcn_body(feature_split):
    def body(src_hbm, dst_hbm, tbl_hbm, out_hbm, stg, rows_a, rows_b,
             is_a, id_a, is_b, id_b, acc, sem_a, sem_b):
        c = lax.axis_index("c")
        s = lax.axis_index("s")

        if feature_split:
            base = s * (NE // NSUB)
            nwin = NE // NSUB // EW
        else:
            base = (c * NSUB + s) * (NE // (NC * NSUB))
            nwin = NE // (NC * NSUB) // EW

        for p in range(2):
            if feature_split:
                tbl_row = tbl_hbm.at[c, p]
                out_row = out_hbm.at[c, p]
            else:
                tbl_row = tbl_hbm.at[p]
                out_row = out_hbm.at[c, p]
            _zero_acc_and_barrier(stg, acc, s)
            _edge_pipeline(src_hbm, dst_hbm, tbl_row, acc, base, nwin,
                           rows_a, rows_b, is_a, id_a, is_b, id_b,
                           sem_a, sem_b)
            _writeback(stg, acc, out_row, s)
    return body


_SC_SCRATCH = [
    pltpu.VMEM((625, PW), jnp.float32),
    pltpu.VMEM((EW, PW), jnp.float32),
    pltpu.VMEM((EW, PW), jnp.float32),
    pltpu.VMEM((EW,), jnp.int32),
    pltpu.VMEM((EW,), jnp.int32),
    pltpu.VMEM((EW,), jnp.int32),
    pltpu.VMEM((EW,), jnp.int32),
    pltpu.VMEM_SHARED((N, PW), jnp.float32),
    pltpu.SemaphoreType.DMA,
    pltpu.SemaphoreType.DMA,
]

_gcn1_kernel = functools.partial(
    pl.kernel,
    out_type=jax.ShapeDtypeStruct((NC, 2, N, PW), jnp.float32),
    mesh=_MESH,
    compiler_params=_SC_PARAMS,
    scratch_types=_SC_SCRATCH,
)(_make_gcn_body(feature_split=True))

_gcn2_kernel = functools.partial(
    pl.kernel,
    out_type=jax.ShapeDtypeStruct((NC, 2, N, PW), jnp.float32),
    mesh=_MESH,
    compiler_params=_SC_PARAMS,
    scratch_types=_SC_SCRATCH,
)(_make_gcn_body(feature_split=False))


RB = 2000
GRID = N // RB


def _dinv_from(degr):
    deg = degr[:, 0] + degr[:, 1] + 1.0
    return lax.rsqrt(deg)[:, None]


def _dense_body(x_r, h0_r, w1_r, b1_r, wih_r, bih_r, whh_r, bhh_r, w2_r, b2_r,
                deg_r, h_o, m_o):
    x = jnp.maximum(jnp.dot(x_r[...], w1_r[...],
                            preferred_element_type=jnp.float32) + b1_r[...], 0.0)
    gi = jnp.dot(x, wih_r[...], preferred_element_type=jnp.float32) + bih_r[...]
    gh = jnp.dot(h0_r[...], whh_r[...], preferred_element_type=jnp.float32) + bhh_r[...]
    r = jax.nn.sigmoid(gi[:, :HF] + gh[:, :HF])
    z = jax.nn.sigmoid(gi[:, HF:2 * HF] + gh[:, HF:2 * HF])
    n = jnp.tanh(gi[:, 2 * HF:] + r * gh[:, 2 * HF:])
    h = (1.0 - z) * n + z * h0_r[...]
    o = jnp.dot(h, w2_r[...], preferred_element_type=jnp.float32) + b2_r[...]
    m = o * _dinv_from(deg_r[...])
    h_o[...] = h
    planes = [m[:, k * PW:(k + 1) * PW] for k in range(4)]
    m_o[...] = jnp.stack(planes, axis=0).reshape(NC, 2, RB, PW)


def _tc_dense(x2d, h02d, w1, b1, wihT, bih, whhT, bhh, w2, b2, degp):
    full = lambda shape: pl.BlockSpec(shape, lambda i: (0,) * len(shape))
    return pl.pallas_call(
        _dense_body,
        grid=(GRID,),
        in_specs=[
            pl.BlockSpec((RB, 128), lambda i: (i, 0)),
            pl.BlockSpec((RB, HF), lambda i: (i, 0)),
            full((128, HF)), full((1, HF)),
            full((HF, 3 * HF)), full((1, 3 * HF)),
            full((HF, 3 * HF)), full((1, 3 * HF)),
            full((HF, HF)), full((1, HF)),
            pl.BlockSpec((RB, NC), lambda i: (i, 0)),
        ],
        out_specs=[
            pl.BlockSpec((RB, HF), lambda i: (i, 0)),
            pl.BlockSpec((NC, 2, RB, PW), lambda i: (0, 0, i, 0)),
        ],
        out_shape=[
            jax.ShapeDtypeStruct((N, HF), jnp.float32),
            jax.ShapeDtypeStruct((NC, 2, N, PW), jnp.float32),
        ],
    )(x2d, h02d, w1, b1, wihT, bih, whhT, bhh, w2, b2, degp)


def _mid_body(acc_r, m_r, deg_r, wg1_r, bg1_r, wg2_r, out_r):
    dinv = _dinv_from(deg_r[...])
    s = jnp.concatenate(
        [acc_r[0, 0] + m_r[0, 0], acc_r[0, 1] + m_r[0, 1],
         acc_r[1, 0] + m_r[1, 0], acc_r[1, 1] + m_r[1, 1]], axis=1) * dinv
    q1 = jnp.maximum(jnp.dot(s, wg1_r[...],
                             preferred_element_type=jnp.float32) + bg1_r[...], 0.0)
    t = jnp.dot(q1, wg2_r[...], preferred_element_type=jnp.float32) * dinv
    out_r[...] = jnp.stack([t[:, :PW], t[:, PW:]], axis=0)


def _tc_mid(acc1, mboth, degp, wg1, bg1, wg2p):
    full = lambda shape: pl.BlockSpec(shape, lambda i: (0,) * len(shape))
    return pl.pallas_call(
        _mid_body,
        grid=(GRID,),
        in_specs=[
            pl.BlockSpec((NC, 2, RB, PW), lambda i: (0, 0, i, 0)),
            pl.BlockSpec((NC, 2, RB, PW), lambda i: (0, 0, i, 0)),
            pl.BlockSpec((RB, NC), lambda i: (i, 0)),
            full((HF, HF)), full((1, HF)), full((HF, 2 * PW)),
        ],
        out_specs=pl.BlockSpec((2, RB, PW), lambda i: (0, i, 0)),
        out_shape=jax.ShapeDtypeStruct((2, N, PW), jnp.float32),
    )(acc1, mboth, degp, wg1, bg1, wg2p)


def _final_body(acc_r, m_r, deg_r, bg2_r, out_r):
    dinv = _dinv_from(deg_r[...])
    q = jnp.concatenate(
        [acc_r[0, 0] + acc_r[1, 0] + m_r[0],
         acc_r[0, 1] + acc_r[1, 1] + m_r[1]], axis=1)
    out_r[...] = q * dinv + bg2_r[...]


def _tc_final(acc2, m2p, degp, bg2p):
    full = lambda shape: pl.BlockSpec(shape, lambda i: (0,) * len(shape))
    return pl.pallas_call(
        _final_body,
        grid=(GRID,),
        in_specs=[
            pl.BlockSpec((NC, 2, RB, PW), lambda i: (0, 0, i, 0)),
            pl.BlockSpec((2, RB, PW), lambda i: (0, i, 0)),
            pl.BlockSpec((RB, NC), lambda i: (i, 0)),
            full((1, 2 * PW)),
        ],
        out_specs=pl.BlockSpec((RB, 2 * PW), lambda i: (i, 0)),
        out_shape=jax.ShapeDtypeStruct((N, 2 * PW), jnp.float32),
    )(acc2, m2p, degp, bg2p)


def kernel(inputs, hidden_state, edge_index, W1, b1, W_ih, W_hh, b_ih, b_hh,
           W2, b2, Wg1, bg1, Wg2, bg2):
    b, a, e = inputs.shape
    x2d = inputs.reshape(N, e)
    h02d = hidden_state.reshape(N, HF)
    src1d = edge_index[0]
    dst1d = edge_index[1]

    wihT = W_ih.T
    whhT = W_hh.T
    wg2p = jnp.pad(Wg2, ((0, 0), (0, 2 * PW - Wg2.shape[1])))
    bg2p = jnp.pad(bg2, (0, 2 * PW - bg2.shape[0])).reshape(1, 2 * PW)
    row = lambda v: v.reshape(1, -1)

    degp = _deg_kernel(dst1d).T
    h, mboth = _tc_dense(x2d, h02d, W1, row(b1), wihT, row(b_ih), whhT,
                         row(b_hh), W2, row(b2), degp)
    acc1 = _gcn1_kernel(src1d, dst1d, mboth)
    m2p = _tc_mid(acc1, mboth, degp, Wg1, row(bg1), wg2p)
    acc2 = _gcn2_kernel(src1d, dst1d, m2p)
    q32 = _tc_final(acc2, m2p, degp, bg2p)
    return q32[:, :20].reshape(b, a, 20), h.reshape(b, a, HF)
